# Initial kernel scaffold; baseline (speedup 1.0000x reference)
import functools

import jax
import jax.numpy as jnp
from jax import lax
from jax.experimental import pallas as pl
from jax.experimental.pallas import tpu as pltpu
from jax.experimental.pallas import tpu_sc as plsc

N = 10000
E = 320000
D = 128
G = 64

NC, NS = 2, 16
NW = NC * NS
EPT = E // NW
CH = 80
NCHUNK = EPT // CH
RPT = N // NS
DEGW = 16
NPAD = N + 16

_mesh = plsc.VectorSubcoreMesh(
    core_axis_name="c", subcore_axis_name="s", num_cores=NC, num_subcores=NS)


def _worker():
    c = lax.axis_index("c")
    s = lax.axis_index("s")
    return c, s, c * NS + s


def _degree_body(dst_hbm, dacc_out, dst_v, ones_v, zrow_v, dacc_sh):
    c, s, w = _worker()

    @pl.loop(0, CH)
    def _(i):
        ones_v[i, :] = jnp.ones((DEGW,), jnp.float32)

    @pl.loop(0, 125)
    def _(i):
        zrow_v[i, :] = jnp.zeros((DEGW,), jnp.float32)

    @pl.loop(0, RPT // 125)
    def _(k):
        pltpu.sync_copy(zrow_v, dacc_sh.at[pl.ds(s * RPT + k * 125, 125)])

    plsc.subcore_barrier()
    base = w * EPT

    @pl.loop(0, NCHUNK)
    def _(i):
        pltpu.sync_copy(dst_hbm.at[pl.ds(base + i * CH, CH)], dst_v)
        pltpu.sync_copy(ones_v, dacc_sh.at[dst_v], add=True)

    plsc.subcore_barrier()
    pltpu.sync_copy(dacc_sh.at[pl.ds(s * RPT, RPT)],
                    dacc_out.at[c, pl.ds(s * RPT, RPT)])


_degree_call = pl.kernel(
    _degree_body,
    out_type=jax.ShapeDtypeStruct((NC, N, DEGW), jnp.float32),
    mesh=_mesh,
    scratch_types=[
        pltpu.VMEM((CH,), jnp.int32),
        pltpu.VMEM((CH, DEGW), jnp.float32),
        pltpu.VMEM((125, DEGW), jnp.float32),
        pltpu.VMEM_SHARED((N, DEGW), jnp.float32),
    ],
)


def _scatter_body(h_hbm, src_hbm, dst_hbm, acc_out,
                  src_v, dst_v, rows_v, zrow_v, acc_sh, sem):
    c, s, w = _worker()

    @pl.loop(0, 125)
    def _(i):
        for j in range(D // 16):
            zrow_v[i, pl.ds(j * 16, 16)] = jnp.zeros((16,), jnp.float32)

    @pl.loop(0, RPT // 125)
    def _(k):
        pltpu.sync_copy(zrow_v, acc_sh.at[pl.ds(s * RPT + k * 125, 125)])

    plsc.subcore_barrier()
    base = w * EPT

    @pl.loop(0, NCHUNK)
    def _(i):
        pltpu.sync_copy(src_hbm.at[pl.ds(base + i * CH, CH)], src_v)
        pltpu.sync_copy(dst_hbm.at[pl.ds(base + i * CH, CH)], dst_v)
        pltpu.async_copy(h_hbm.at[src_v], rows_v, sem).wait()
        pltpu.sync_copy(rows_v, acc_sh.at[dst_v], add=True)

    plsc.subcore_barrier()
    pltpu.sync_copy(acc_sh.at[pl.ds(s * RPT, RPT)],
                    acc_out.at[c, pl.ds(s * RPT, RPT)])


_scatter_call = pl.kernel(
    _scatter_body,
    out_type=jax.ShapeDtypeStruct((NC, N, D), jnp.float32),
    mesh=_mesh,
    scratch_types=[
        pltpu.VMEM((CH,), jnp.int32),
        pltpu.VMEM((CH,), jnp.int32),
        pltpu.VMEM((CH, D), jnp.float32),
        pltpu.VMEM((125, D), jnp.float32),
        pltpu.VMEM_SHARED((N, D), jnp.float32),
        pltpu.SemaphoreType.DMA,
    ],
)


def _pool_body(h_hbm, batch_hbm, out_hbm, batch_v, row_v, cnt_v, rowout_v):
    c, s, w = _worker()
    pltpu.sync_copy(batch_hbm, batch_v)

    g0 = 2 * w
    g0v = jnp.full((16,), 0, jnp.int32) + g0
    g1v = g0v + 1
    g2v = g0v + 2

    def count_body(k, carry):
        lo, mid, hi = carry
        bv = batch_v[pl.ds(k * 16, 16)]
        lo = lo + plsc.all_reduce_population_count(bv < g0v)
        mid = mid + plsc.all_reduce_population_count(bv < g1v)
        hi = hi + plsc.all_reduce_population_count(bv < g2v)
        return lo, mid, hi

    z = jnp.zeros((16,), jnp.int32)
    lo, mid, hi = lax.fori_loop(0, N // 16, count_body, (z, z, z))
    cnt_v[pl.ds(0, 16)] = lo
    cnt_v[pl.ds(16, 16)] = mid
    cnt_v[pl.ds(32, 16)] = hi
    s_lo = cnt_v[0]
    s_mid = cnt_v[16]
    s_hi = cnt_v[32]

    neg_inf = jnp.full((16,), -jnp.inf, jnp.float32)
    zf = jnp.zeros((16,), jnp.float32)

    @pl.loop(0, 2)
    def _(t):
        start = jnp.where(t == 0, s_lo, s_mid)
        end = jnp.where(t == 0, s_mid, s_hi)
        cnt = end - start
        cnt_splat = jnp.full((16,), 0, jnp.int32) + cnt
        nch = (cnt + 15) // 16

        def chunk_body(cidx, carry):
            accs = list(carry)
            pltpu.sync_copy(h_hbm.at[pl.ds(start + cidx * 16, 16)], row_v)
            for i in range(16):
                rid = jnp.full((16,), 0, jnp.int32) + (cidx * 16 + i)
                pv = rid < cnt_splat
                for j in range(D // 16):
                    r = row_v[i, pl.ds(j * 16, 16)]
                    accs[j] = accs[j] + jnp.where(pv, r, zf)
                    accs[8 + j] = jnp.maximum(accs[8 + j],
                                              jnp.where(pv, r, neg_inf))
            return tuple(accs)

        init = tuple([zf] * 8 + [neg_inf] * 8)
        accs = lax.fori_loop(0, nch, chunk_body, init)

        inv = 1.0 / jnp.maximum(cnt, 1).astype(jnp.float32)
        for j in range(D // 16):
            rowout_v[pl.ds(j * 16, 16)] = accs[j]
            rowout_v[pl.ds(D + j * 16, 16)] = accs[8 + j]
            rowout_v[pl.ds(2 * D + j * 16, 16)] = accs[j] * inv
        pltpu.sync_copy(rowout_v, out_hbm.at[g0 + t])


_pool_call = pl.kernel(
    _pool_body,
    out_type=jax.ShapeDtypeStruct((G, 3 * D), jnp.float32),
    mesh=_mesh,
    scratch_types=[
        pltpu.VMEM((N,), jnp.int32),
        pltpu.VMEM((16, D), jnp.float32),
        pltpu.VMEM((48,), jnp.int32),
        pltpu.VMEM((3 * D,), jnp.float32),
    ],
)


BR = 1000


def _dinv_of(da0_ref, da1_ref):
    deg = 1.0 + da0_ref[:, 0] + da1_ref[:, 0]
    return lax.rsqrt(deg)


def _mm1_body(x_ref, w_ref, da0_ref, da1_ref, o_ref):
    dinv = _dinv_of(da0_ref, da1_ref)
    t = jnp.dot(x_ref[...], w_ref[...], preferred_element_type=jnp.float32)
    o_ref[...] = t * dinv[:, None]


def _fuse_mm_body(a0_ref, a1_ref, hp_ref, da0_ref, da1_ref, b_ref, w_ref,
                  o_ref):
    dinv = _dinv_of(da0_ref, da1_ref)
    pre = dinv[:, None] * (a0_ref[...] + a1_ref[...] + hp_ref[...])
    h = jnp.maximum(pre + b_ref[...], 0.0)
    t = jnp.dot(h, w_ref[...], preferred_element_type=jnp.float32)
    o_ref[...] = t * dinv[:, None]


def _fuse_body(a0_ref, a1_ref, hp_ref, da0_ref, da1_ref, b_ref, o_ref):
    dinv = _dinv_of(da0_ref, da1_ref)
    pre = dinv[:, None] * (a0_ref[...] + a1_ref[...] + hp_ref[...])
    o_ref[...] = jnp.maximum(pre + b_ref[...], 0.0)


_row_spec = pl.BlockSpec((BR, D), lambda i: (i, 0))
_deg_spec = pl.BlockSpec((BR, DEGW), lambda i: (i, 0))
_w_spec = pl.BlockSpec((D, D), lambda i: (0, 0))
_b_spec = pl.BlockSpec((1, D), lambda i: (0, 0))
_out_shape_nd = jax.ShapeDtypeStruct((N, D), jnp.float32)

_mm1_call = pl.pallas_call(
    _mm1_body, grid=(N // BR,),
    in_specs=[_row_spec, _w_spec, _deg_spec, _deg_spec],
    out_specs=_row_spec, out_shape=_out_shape_nd)

_fuse_mm_call = pl.pallas_call(
    _fuse_mm_body, grid=(N // BR,),
    in_specs=[_row_spec, _row_spec, _row_spec, _deg_spec, _deg_spec,
              _b_spec, _w_spec],
    out_specs=_row_spec, out_shape=_out_shape_nd)

_fuse_call = pl.pallas_call(
    _fuse_body, grid=(N // BR,),
    in_specs=[_row_spec, _row_spec, _row_spec, _deg_spec, _deg_spec, _b_spec],
    out_specs=_row_spec, out_shape=_out_shape_nd)


def kernel(x, edge_index, batch, ptr, W1, b1, W2, b2):
    src = edge_index[0]
    dst = edge_index[1]

    dacc = _degree_call(dst)
    da0, da1 = dacc[0], dacc[1]

    h1p = _mm1_call(x, W1, da0, da1)
    a1 = _scatter_call(h1p, src, dst)
    h2p = _fuse_mm_call(a1[0], a1[1], h1p, da0, da1, b1.reshape(1, D), W2)
    a2 = _scatter_call(h2p, src, dst)
    h2 = _fuse_call(a2[0], a2[1], h2p, da0, da1, b2.reshape(1, D))

    h2pad = jnp.pad(h2, ((0, NPAD - N), (0, 0)))
    out = _pool_call(h2pad, batch)
    return out


# speedup vs baseline: 11.7752x; 11.7752x over previous
import functools

import jax
import jax.numpy as jnp
from jax import lax
from jax.experimental import pallas as pl
from jax.experimental.pallas import tpu as pltpu
from jax.experimental.pallas import tpu_sc as plsc

N = 10000
E = 320000
D = 128
G = 64

NC, NS = 2, 16
NW = NC * NS
EPT = E // NW
CH = 80
NCHUNK = EPT // CH
RPT = 632
RPT_LAST = N - 15 * RPT
DEGW = 128
NPAD = N + 32

_mesh = plsc.VectorSubcoreMesh(
    core_axis_name="c", subcore_axis_name="s", num_cores=NC, num_subcores=NS)


def _worker():
    c = lax.axis_index("c")
    s = lax.axis_index("s")
    return c, s, c * NS + s


def _per_subcore_rows(s, run):
    @pl.when(s < NS - 1)
    def _():
        run(RPT)

    @pl.when(s == NS - 1)
    def _():
        run(RPT_LAST)


def _zero_shared(acc_sh, s, zrow_v):
    base = pl.multiple_of(s * RPT, 8)

    def run(nrows):
        @pl.loop(0, nrows // 8)
        def _(k):
            pltpu.sync_copy(zrow_v, acc_sh.at[pl.ds(base + k * 8, 8)])

    _per_subcore_rows(s, run)


def _copyout_shared(acc_sh, acc_out, c, s):
    base = pl.multiple_of(s * RPT, 8)

    def run(nrows):
        pltpu.sync_copy(acc_sh.at[pl.ds(base, nrows)],
                        acc_out.at[c, pl.ds(base, nrows)])

    _per_subcore_rows(s, run)


def _degree_body(dst_hbm, dacc_out, dst_v, ones_v, zrow_v, dacc_sh):
    c, s, w = _worker()

    @pl.loop(0, CH)
    def _(i):
        for j in range(DEGW // 16):
            ones_v[i, pl.ds(j * 16, 16)] = jnp.ones((16,), jnp.float32)

    @pl.loop(0, 8)
    def _(i):
        for j in range(DEGW // 16):
            zrow_v[i, pl.ds(j * 16, 16)] = jnp.zeros((16,), jnp.float32)

    _zero_shared(dacc_sh, s, zrow_v)
    plsc.subcore_barrier()
    base = w * EPT

    @pl.loop(0, NCHUNK)
    def _(i):
        pltpu.sync_copy(dst_hbm.at[pl.ds(base + i * CH, CH)], dst_v)
        pltpu.sync_copy(ones_v, dacc_sh.at[dst_v], add=True)

    plsc.subcore_barrier()
    _copyout_shared(dacc_sh, dacc_out, c, s)


_degree_call = pl.kernel(
    _degree_body,
    out_type=jax.ShapeDtypeStruct((NC, N, DEGW), jnp.float32),
    mesh=_mesh,
    compiler_params=pltpu.CompilerParams(needs_layout_passes=False),
    scratch_types=[
        pltpu.VMEM((CH,), jnp.int32),
        pltpu.VMEM((CH, DEGW), jnp.float32),
        pltpu.VMEM((8, DEGW), jnp.float32),
        pltpu.VMEM_SHARED((N, DEGW), jnp.float32),
    ],
)


def _scatter_body(h_hbm, src_hbm, dst_hbm, acc_out,
                  src_v, dst_v, rows_v, zrow_v, acc_sh, sem):
    c, s, w = _worker()

    @pl.loop(0, 8)
    def _(i):
        for j in range(D // 16):
            zrow_v[i, pl.ds(j * 16, 16)] = jnp.zeros((16,), jnp.float32)

    _zero_shared(acc_sh, s, zrow_v)
    plsc.subcore_barrier()
    base = w * EPT

    @pl.loop(0, NCHUNK)
    def _(i):
        pltpu.sync_copy(src_hbm.at[pl.ds(base + i * CH, CH)], src_v)
        pltpu.sync_copy(dst_hbm.at[pl.ds(base + i * CH, CH)], dst_v)
        pltpu.async_copy(h_hbm.at[src_v], rows_v, sem).wait()
        pltpu.sync_copy(rows_v, acc_sh.at[dst_v], add=True)

    plsc.subcore_barrier()
    _copyout_shared(acc_sh, acc_out, c, s)


_scatter_call = pl.kernel(
    _scatter_body,
    out_type=jax.ShapeDtypeStruct((NC, N, D), jnp.float32),
    mesh=_mesh,
    compiler_params=pltpu.CompilerParams(needs_layout_passes=False),
    scratch_types=[
        pltpu.VMEM((CH,), jnp.int32),
        pltpu.VMEM((CH,), jnp.int32),
        pltpu.VMEM((CH, D), jnp.float32),
        pltpu.VMEM((8, D), jnp.float32),
        pltpu.VMEM_SHARED((N, D), jnp.float32),
        pltpu.SemaphoreType.DMA,
    ],
)


def _pool_body(h_hbm, batch_hbm, out_hbm, batch_v, row_v, rowout_v):
    c, s, w = _worker()
    pltpu.sync_copy(batch_hbm, batch_v)

    g0 = 2 * w
    g0v = jnp.full((16,), 0, jnp.int32) + g0
    g1v = g0v + 1
    g2v = g0v + 2

    one = jnp.ones((16,), jnp.int32)
    zi = jnp.zeros((16,), jnp.int32)

    def count_body(k, carry):
        lo, mid, hi = carry
        bv = batch_v[pl.ds(k * 16, 16)]
        lo = lo + jnp.sum(jnp.where(bv < g0v, one, zi))
        mid = mid + jnp.sum(jnp.where(bv < g1v, one, zi))
        hi = hi + jnp.sum(jnp.where(bv < g2v, one, zi))
        return lo, mid, hi

    z0 = jnp.int32(0)
    s_lo, s_mid, s_hi = lax.fori_loop(0, N // 16, count_body, (z0, z0, z0))

    neg_inf = jnp.full((16,), -jnp.inf, jnp.float32)
    zf = jnp.zeros((16,), jnp.float32)

    @pl.loop(0, 2)
    def _(t):
        start = jnp.where(t == 0, s_lo, s_mid)
        end = jnp.where(t == 0, s_mid, s_hi)
        cnt = end - start
        abase = pl.multiple_of((start // 16) * 16, 16)
        nch = (end - abase + 15) // 16
        start_splat = jnp.full((16,), 0, jnp.int32) + start
        end_splat = jnp.full((16,), 0, jnp.int32) + end

        def chunk_body(cidx, carry):
            accs = list(carry)
            rbase = pl.multiple_of(abase + cidx * 16, 16)
            pltpu.sync_copy(h_hbm.at[pl.ds(rbase, 16)], row_v)
            for i in range(16):
                rid = jnp.full((16,), 0, jnp.int32) + (rbase + i)
                pv = (rid >= start_splat) & (rid < end_splat)
                for j in range(D // 16):
                    r = row_v[i, pl.ds(j * 16, 16)]
                    accs[j] = accs[j] + jnp.where(pv, r, zf)
                    accs[8 + j] = jnp.maximum(accs[8 + j],
                                              jnp.where(pv, r, neg_inf))
            return tuple(accs)

        init = tuple([zf] * 8 + [neg_inf] * 8)
        accs = lax.fori_loop(0, nch, chunk_body, init)

        cnt_splat = end_splat - start_splat
        onei = jnp.full((16,), 1, jnp.int32)
        inv = jnp.ones((16,), jnp.float32) / (
            jnp.maximum(cnt_splat, onei).astype(jnp.float32))
        for j in range(D // 16):
            rowout_v[0, pl.ds(j * 16, 16)] = accs[j]
            rowout_v[1, pl.ds(j * 16, 16)] = accs[8 + j]
            rowout_v[2, pl.ds(j * 16, 16)] = accs[j] * inv
        pltpu.sync_copy(rowout_v, out_hbm.at[w, t])


_pool_call = pl.kernel(
    _pool_body,
    out_type=jax.ShapeDtypeStruct((NW, 2, 3, D), jnp.float32),
    mesh=_mesh,
    compiler_params=pltpu.CompilerParams(needs_layout_passes=False),
    scratch_types=[
        pltpu.VMEM((N,), jnp.int32),
        pltpu.VMEM((16, D), jnp.float32),
        pltpu.VMEM((3, D), jnp.float32),
    ],
)


BR = 1000


def _dinv_of(da0_ref, da1_ref):
    deg = 1.0 + da0_ref[:, 0] + da1_ref[:, 0]
    return lax.rsqrt(deg)


def _mm1_body(x_ref, w_ref, da0_ref, da1_ref, o_ref):
    dinv = _dinv_of(da0_ref, da1_ref)
    t = jnp.dot(x_ref[...], w_ref[...], preferred_element_type=jnp.float32)
    o_ref[...] = t * dinv[:, None]


def _fuse_mm_body(a0_ref, a1_ref, hp_ref, da0_ref, da1_ref, b_ref, w_ref,
                  o_ref):
    dinv = _dinv_of(da0_ref, da1_ref)
    pre = dinv[:, None] * (a0_ref[...] + a1_ref[...] + hp_ref[...])
    h = jnp.maximum(pre + b_ref[...], 0.0)
    t = jnp.dot(h, w_ref[...], preferred_element_type=jnp.float32)
    o_ref[...] = t * dinv[:, None]


def _fuse_body(a0_ref, a1_ref, hp_ref, da0_ref, da1_ref, b_ref, o_ref):
    dinv = _dinv_of(da0_ref, da1_ref)
    pre = dinv[:, None] * (a0_ref[...] + a1_ref[...] + hp_ref[...])
    o_ref[...] = jnp.maximum(pre + b_ref[...], 0.0)


_row_spec = pl.BlockSpec((BR, D), lambda i: (i, 0))
_deg_spec = pl.BlockSpec((BR, DEGW), lambda i: (i, 0))
_w_spec = pl.BlockSpec((D, D), lambda i: (0, 0))
_b_spec = pl.BlockSpec((1, D), lambda i: (0, 0))
_out_shape_nd = jax.ShapeDtypeStruct((N, D), jnp.float32)

_mm1_call = pl.pallas_call(
    _mm1_body, grid=(N // BR,),
    in_specs=[_row_spec, _w_spec, _deg_spec, _deg_spec],
    out_specs=_row_spec, out_shape=_out_shape_nd)

_fuse_mm_call = pl.pallas_call(
    _fuse_mm_body, grid=(N // BR,),
    in_specs=[_row_spec, _row_spec, _row_spec, _deg_spec, _deg_spec,
              _b_spec, _w_spec],
    out_specs=_row_spec, out_shape=_out_shape_nd)

_fuse_call = pl.pallas_call(
    _fuse_body, grid=(N // BR,),
    in_specs=[_row_spec, _row_spec, _row_spec, _deg_spec, _deg_spec, _b_spec],
    out_specs=_row_spec, out_shape=_out_shape_nd)


def kernel(x, edge_index, batch, ptr, W1, b1, W2, b2):
    src = edge_index[0]
    dst = edge_index[1]

    dacc = _degree_call(dst)
    da0, da1 = dacc[0], dacc[1]

    h1p = _mm1_call(x, W1, da0, da1)
    a1 = _scatter_call(h1p, src, dst)
    h2p = _fuse_mm_call(a1[0], a1[1], h1p, da0, da1, b1.reshape(1, D), W2)
    a2 = _scatter_call(h2p, src, dst)
    h2 = _fuse_call(a2[0], a2[1], h2p, da0, da1, b2.reshape(1, D))

    h2pad = jnp.pad(h2, ((0, NPAD - N), (0, 0)))
    out = _pool_call(h2pad, batch)
    return out.reshape(G, 3 * D)
